# baseline (speedup 1.0000x reference)
import functools

import jax
import jax.numpy as jnp
from jax import lax
from jax.experimental import pallas as pl
from jax.experimental.pallas import tpu as pltpu
from jax.experimental.pallas import tpu_sc as plsc

_B = 16384
_D = 32
_H1 = 64
_NC = 2
_NS = 16
_NW = _NC * _NS
_BPW = _B // _NW
_CHUNK = 128
_NCH = _BPW // _CHUNK


@functools.cache
def _gather_kernel():
    mesh = plsc.VectorSubcoreMesh(core_axis_name="c", subcore_axis_name="s")

    @functools.partial(
        pl.kernel,
        mesh=mesh,
        compiler_params=pltpu.CompilerParams(use_tc_tiling_on_sc=False),
        out_type=(
            jax.ShapeDtypeStruct((_B, _D), jnp.float32),
            jax.ShapeDtypeStruct((_B, _D), jnp.float32),
        ),
        scratch_types=[
            pltpu.VMEM((_NCH, _CHUNK), jnp.int32),
            pltpu.VMEM((_NCH, _CHUNK), jnp.int32),
            pltpu.VMEM((_BPW, _D), jnp.float32),
            pltpu.VMEM((_BPW, _D), jnp.float32),
            pltpu.SemaphoreType.DMA,
            pltpu.SemaphoreType.DMA,
        ],
    )
    def gather(users_hbm, items_hbm, utab_hbm, itab_hbm, xu_hbm, xi_hbm,
               uidx_v, iidx_v, urows_v, irows_v, sem_u, sem_i):
        wid = lax.axis_index("s") * _NC + lax.axis_index("c")
        base = wid * _BPW
        pltpu.sync_copy(users_hbm.at[wid], uidx_v)
        pltpu.sync_copy(items_hbm.at[wid], iidx_v)
        ucopies = []
        icopies = []
        for j in range(_NCH):
            dst = pl.ds(j * _CHUNK, _CHUNK)
            ucopies.append(
                pltpu.async_copy(utab_hbm.at[uidx_v.at[j]], urows_v.at[dst], sem_u))
            icopies.append(
                pltpu.async_copy(itab_hbm.at[iidx_v.at[j]], irows_v.at[dst], sem_i))
        for c in ucopies:
            c.wait()
        pltpu.sync_copy(urows_v, xu_hbm.at[pl.ds(base, _BPW)])
        for c in icopies:
            c.wait()
        pltpu.sync_copy(irows_v, xi_hbm.at[pl.ds(base, _BPW)])

    return gather


def _mlp_body(xu_ref, xi_ref, w1u_ref, w1i_ref, b1_ref, w2_ref, b2_ref, out_ref):
    h = (jnp.dot(xu_ref[...], w1u_ref[...], preferred_element_type=jnp.float32)
         + jnp.dot(xi_ref[...], w1i_ref[...], preferred_element_type=jnp.float32)
         + b1_ref[...])
    h = jnp.maximum(h, 0.0)
    out_ref[...] = (
        jnp.dot(h, w2_ref[...], preferred_element_type=jnp.float32) + b2_ref[...])


@functools.cache
def _mlp_call(blk):
    grid = (_B // blk,)
    return pl.pallas_call(
        _mlp_body,
        grid=grid,
        in_specs=[
            pl.BlockSpec((blk, _D), lambda i: (i, 0)),
            pl.BlockSpec((blk, _D), lambda i: (i, 0)),
            pl.BlockSpec((_D, _H1), lambda i: (0, 0)),
            pl.BlockSpec((_D, _H1), lambda i: (0, 0)),
            pl.BlockSpec((1, _H1), lambda i: (0, 0)),
            pl.BlockSpec((_H1, 1), lambda i: (0, 0)),
            pl.BlockSpec((1, 1), lambda i: (0, 0)),
        ],
        out_specs=pl.BlockSpec((blk, 1), lambda i: (i, 0)),
        out_shape=jax.ShapeDtypeStruct((_B, 1), jnp.float32),
    )


def kernel(users, items, user_table, item_table, W1, b1, W2, b2):
    users_r = users.reshape(_NW, _NCH, _CHUNK)
    items_r = items.reshape(_NW, _NCH, _CHUNK)
    xu, xi = _gather_kernel()(users_r, items_r, user_table, item_table)
    w1u_t = W1[:, :_D].T
    w1i_t = W1[:, _D:].T
    return _mlp_call(2048)(
        xu, xi, w1u_t, w1i_t, b1.reshape(1, _H1), W2.T, b2.reshape(1, 1))


# speedup vs baseline: 1.0107x; 1.0107x over previous
import functools

import jax
import jax.numpy as jnp
from jax import lax
from jax.experimental import pallas as pl
from jax.experimental.pallas import tpu as pltpu
from jax.experimental.pallas import tpu_sc as plsc

_B = 16384
_D = 32
_H1 = 64
_LANES = 128
_RPS = _LANES // _D * _D // _D
_NC = 2
_NS = 16
_NW = _NC * _NS
_BPW = _B // _NW
_CHUNK = 128
_NCH = _BPW // _CHUNK


@functools.cache
def _gather_kernel():
    mesh = plsc.VectorSubcoreMesh(core_axis_name="c", subcore_axis_name="s")

    @functools.partial(
        pl.kernel,
        mesh=mesh,
        out_type=(
            jax.ShapeDtypeStruct((_B, _LANES), jnp.float32),
            jax.ShapeDtypeStruct((_B, _LANES), jnp.float32),
        ),
        scratch_types=[
            pltpu.VMEM((_NCH, _CHUNK), jnp.int32),
            pltpu.VMEM((_NCH, _CHUNK), jnp.int32),
            pltpu.VMEM((_CHUNK, _LANES), jnp.float32),
            pltpu.VMEM((_CHUNK, _LANES), jnp.float32),
            pltpu.SemaphoreType.DMA,
            pltpu.SemaphoreType.DMA,
        ],
    )
    def gather(users_hbm, items_hbm, utab_hbm, itab_hbm, xu_hbm, xi_hbm,
               uidx_v, iidx_v, urows_v, irows_v, sem_u, sem_i):
        wid = lax.axis_index("s") * _NC + lax.axis_index("c")
        base = wid * _BPW
        pltpu.sync_copy(users_hbm.at[wid], uidx_v)
        pltpu.sync_copy(items_hbm.at[wid], iidx_v)
        for j in range(_NCH):
            cu = pltpu.async_copy(utab_hbm.at[uidx_v.at[j]], urows_v, sem_u)
            ci = pltpu.async_copy(itab_hbm.at[iidx_v.at[j]], irows_v, sem_i)
            dst = pl.ds(base + j * _CHUNK, _CHUNK)
            cu.wait()
            pltpu.sync_copy(urows_v, xu_hbm.at[dst])
            ci.wait()
            pltpu.sync_copy(irows_v, xi_hbm.at[dst])

    return gather


def _mlp_body(xu_ref, xi_ref, offu_ref, offi_ref, w1u_ref, w1i_ref, b1_ref,
              w2_ref, b2_ref, out_ref):
    blk = xu_ref.shape[0]
    grp = lax.broadcasted_iota(jnp.int32, (blk, _LANES), 1) // _D
    xu = jnp.where(grp == offu_ref[...], xu_ref[...], 0.0)
    xi = jnp.where(grp == offi_ref[...], xi_ref[...], 0.0)
    h = (jnp.dot(xu, w1u_ref[...], preferred_element_type=jnp.float32)
         + jnp.dot(xi, w1i_ref[...], preferred_element_type=jnp.float32)
         + b1_ref[...])
    h = jnp.maximum(h, 0.0)
    out_ref[...] = (
        jnp.dot(h, w2_ref[...], preferred_element_type=jnp.float32) + b2_ref[...])


@functools.cache
def _mlp_call(blk):
    grid = (_B // blk,)
    return pl.pallas_call(
        _mlp_body,
        grid=grid,
        in_specs=[
            pl.BlockSpec((blk, _LANES), lambda i: (i, 0)),
            pl.BlockSpec((blk, _LANES), lambda i: (i, 0)),
            pl.BlockSpec((blk, 1), lambda i: (i, 0)),
            pl.BlockSpec((blk, 1), lambda i: (i, 0)),
            pl.BlockSpec((_LANES, _H1), lambda i: (0, 0)),
            pl.BlockSpec((_LANES, _H1), lambda i: (0, 0)),
            pl.BlockSpec((1, _H1), lambda i: (0, 0)),
            pl.BlockSpec((_H1, 1), lambda i: (0, 0)),
            pl.BlockSpec((1, 1), lambda i: (0, 0)),
        ],
        out_specs=pl.BlockSpec((blk, 1), lambda i: (i, 0)),
        out_shape=jax.ShapeDtypeStruct((_B, 1), jnp.float32),
    )


def kernel(users, items, user_table, item_table, W1, b1, W2, b2):
    n_super = user_table.shape[0] // _RPS
    utab = user_table.reshape(n_super, _LANES)
    itab = item_table.reshape(n_super, _LANES)
    su = (users // _RPS).reshape(_NW, _NCH, _CHUNK)
    si = (items // _RPS).reshape(_NW, _NCH, _CHUNK)
    xu128, xi128 = _gather_kernel()(su, si, utab, itab)
    offu = (users % _RPS).reshape(_B, 1)
    offi = (items % _RPS).reshape(_B, 1)
    w1u_t = jnp.concatenate([W1[:, :_D].T] * _RPS, axis=0)
    w1i_t = jnp.concatenate([W1[:, _D:].T] * _RPS, axis=0)
    return _mlp_call(2048)(
        xu128, xi128, offu, offi, w1u_t, w1i_t,
        b1.reshape(1, _H1), W2.T, b2.reshape(1, 1))


# speedup vs baseline: 3.6597x; 3.6208x over previous
import functools

import jax
import jax.numpy as jnp
from jax import lax
from jax.experimental import pallas as pl
from jax.experimental.pallas import tpu as pltpu
from jax.experimental.pallas import tpu_sc as plsc

_B = 16384
_D = 32
_H1 = 64
_LANES = 128
_NC = 2
_NS = 16
_NW = _NC * _NS
_BPW = _B // _NW
_K = 4
_GRP = 16
_NGR = _BPW // _GRP
_CH = 128


@functools.cache
def _gather_kernel():
    mesh = plsc.VectorSubcoreMesh(core_axis_name="c", subcore_axis_name="s")

    @functools.partial(
        pl.kernel,
        mesh=mesh,
        compiler_params=pltpu.CompilerParams(needs_layout_passes=False),
        out_type=(
            jax.ShapeDtypeStruct((_B, _D), jnp.float32),
            jax.ShapeDtypeStruct((_B, _D), jnp.float32),
        ),
        scratch_types=(
            [pltpu.VMEM((_NGR, _GRP), jnp.int32),
             pltpu.VMEM((_NGR, _GRP), jnp.int32),
             pltpu.VMEM((_K, _D, _LANES), jnp.float32),
             pltpu.VMEM((_K, _D, _LANES), jnp.float32),
             pltpu.VMEM((_CH, _D), jnp.float32),
             pltpu.VMEM((_CH, _D), jnp.float32)]
            + [pltpu.SemaphoreType.DMA] * (2 * _K)
        ),
    )
    def gather(users_hbm, items_hbm, utabT, itabT, xu_hbm, xi_hbm,
               uidx_v, iidx_v, ublk, iblk, uout, iout, *sems):
        usem = sems[:_K]
        isem = sems[_K:]
        wid = lax.axis_index("s") * _NC + lax.axis_index("c")
        base = wid * _BPW
        pltpu.sync_copy(users_hbm.at[wid], uidx_v)
        pltpu.sync_copy(items_hbm.at[wid], iidx_v)
        rows0 = lax.broadcasted_iota(jnp.int32, (16,), 0)
        zeros = jnp.zeros((16,), jnp.int32)

        def scalar_idx(idx_v, r):
            v = idx_v[r >> 4]
            return lax.reduce_sum(jnp.where(rows0 == (r & 15), v, zeros),
                                  axes=(0,))

        def fire(tab, idx_v, blk, sem, r, b):
            i = scalar_idx(idx_v, r)
            off = pl.multiple_of((i >> 7) * _LANES, _LANES)
            pltpu.async_copy(tab.at[:, pl.ds(off, _LANES)], blk.at[b], sem[b])

        def wait(tab, blk, sem, b):
            pltpu.make_async_copy(tab.at[:, pl.ds(0, _LANES)], blk.at[b],
                                  sem[b]).wait()

        def extract(idx_v, blk, out_v, r, b):
            i = scalar_idx(idx_v, r)
            lane = jnp.full((16,), i & (_LANES - 1), jnp.int32)
            g0 = plsc.load_gather(blk.at[b], [rows0, lane])
            g1 = plsc.load_gather(blk.at[b], [rows0 + 16, lane])
            rr = jnp.full((16,), r & (_CH - 1), jnp.int32)
            plsc.store_scatter(out_v, [rr, rows0], g0)
            plsc.store_scatter(out_v, [rr, rows0 + 16], g1)

        for k in range(_K):
            fire(utabT, uidx_v, ublk, usem, k, k)
            fire(itabT, iidx_v, iblk, isem, k, k)

        nwave = _BPW // _K

        def body(w, _):
            for k in range(_K):
                r = w * _K + k
                wait(utabT, ublk, usem, k)
                extract(uidx_v, ublk, uout, r, k)
                wait(itabT, iblk, isem, k)
                extract(iidx_v, iblk, iout, r, k)

                @pl.when(w + 1 < nwave)
                def _():
                    fire(utabT, uidx_v, ublk, usem, r + _K, k)
                    fire(itabT, iidx_v, iblk, isem, r + _K, k)

            @pl.when((w & ((_CH // _K) - 1)) == (_CH // _K) - 1)
            def _():
                coff = (w >> 5) * _CH
                pltpu.sync_copy(uout, xu_hbm.at[pl.ds(base + coff, _CH)])
                pltpu.sync_copy(iout, xi_hbm.at[pl.ds(base + coff, _CH)])
            return 0

        lax.fori_loop(0, nwave, body, 0)

    return gather


def _mlp_body(xu_ref, xi_ref, w1u_ref, w1i_ref, b1_ref, w2_ref, b2_ref,
              out_ref):
    h = (jnp.dot(xu_ref[...], w1u_ref[...], preferred_element_type=jnp.float32)
         + jnp.dot(xi_ref[...], w1i_ref[...], preferred_element_type=jnp.float32)
         + b1_ref[...])
    h = jnp.maximum(h, 0.0)
    out_ref[...] = (
        jnp.dot(h, w2_ref[...], preferred_element_type=jnp.float32) + b2_ref[...])


@functools.cache
def _mlp_call(blk):
    grid = (_B // blk,)
    return pl.pallas_call(
        _mlp_body,
        grid=grid,
        in_specs=[
            pl.BlockSpec((blk, _D), lambda i: (i, 0)),
            pl.BlockSpec((blk, _D), lambda i: (i, 0)),
            pl.BlockSpec((_D, _H1), lambda i: (0, 0)),
            pl.BlockSpec((_D, _H1), lambda i: (0, 0)),
            pl.BlockSpec((1, _H1), lambda i: (0, 0)),
            pl.BlockSpec((_H1, 1), lambda i: (0, 0)),
            pl.BlockSpec((1, 1), lambda i: (0, 0)),
        ],
        out_specs=pl.BlockSpec((blk, 1), lambda i: (i, 0)),
        out_shape=jax.ShapeDtypeStruct((_B, 1), jnp.float32),
    )


def kernel(users, items, user_table, item_table, W1, b1, W2, b2):
    users_r = users.reshape(_NW, _NGR, _GRP)
    items_r = items.reshape(_NW, _NGR, _GRP)
    xu, xi = _gather_kernel()(users_r, items_r, user_table.T, item_table.T)
    w1u_t = W1[:, :_D].T
    w1i_t = W1[:, _D:].T
    return _mlp_call(2048)(
        xu, xi, w1u_t, w1i_t, b1.reshape(1, _H1), W2.T, b2.reshape(1, 1))
